# Initial kernel scaffold; baseline (speedup 1.0000x reference)
import jax
import jax.numpy as jnp
from jax.experimental import pallas as pl


def kernel(in_feat, edge_index, W1, b1, W2, b2):
    raise NotImplementedError("write your pallas kernel here")


# speedup vs baseline: 5.6436x; 5.6436x over previous
import functools

import jax
import jax.numpy as jnp
from jax import lax
from jax.experimental import pallas as pl
from jax.experimental.pallas import tpu as pltpu
from jax.experimental.pallas import tpu_sc as plsc

N = 10000
D = 128
E = 320000
NC = 2
NS = 16
NW = NC * NS
EPT = E // NW
K = 80
NCHUNK = EPT // K
NPAD = 10240
STRIPE = NPAD // NS
BR = 1000

_mesh = plsc.VectorSubcoreMesh(
    core_axis_name="c", subcore_axis_name="s", num_cores=NC, num_subcores=NS
)


@functools.partial(
    pl.kernel,
    out_type=jax.ShapeDtypeStruct((NC, 2, NPAD), jnp.float32),
    mesh=_mesh,
    scratch_types=[
        pltpu.VMEM((K,), jnp.int32),
        pltpu.VMEM((K,), jnp.int32),
        pltpu.VMEM((K,), jnp.float32),
        pltpu.VMEM((STRIPE,), jnp.float32),
        pltpu.VMEM_SHARED((NPAD,), jnp.float32),
        pltpu.VMEM_SHARED((NPAD,), jnp.float32),
    ],
)
def _deg_kernel(src_hbm, dst_hbm, deg_out, sidx, didx, ones_v, zb, od_sh, id_sh):
    c = lax.axis_index("c")
    s = lax.axis_index("s")
    wid = s * NC + c

    for j in range(K // 16):
        ones_v[pl.ds(j * 16, 16)] = jnp.ones((16,), jnp.float32)

    def zbody(i, _):
        zb[pl.ds(i * 16, 16)] = jnp.zeros((16,), jnp.float32)
        return 0

    lax.fori_loop(0, STRIPE // 16, zbody, 0)
    pltpu.sync_copy(zb, od_sh.at[pl.ds(s * STRIPE, STRIPE)])
    pltpu.sync_copy(zb, id_sh.at[pl.ds(s * STRIPE, STRIPE)])
    plsc.subcore_barrier()

    def body(i, _):
        base = wid * EPT + i * K
        pltpu.sync_copy(src_hbm.at[pl.ds(base, K)], sidx)
        pltpu.sync_copy(dst_hbm.at[pl.ds(base, K)], didx)
        pltpu.sync_copy(ones_v, od_sh.at[sidx], add=True)
        pltpu.sync_copy(ones_v, id_sh.at[didx], add=True)
        return 0

    lax.fori_loop(0, NCHUNK, body, 0)
    plsc.subcore_barrier()

    pltpu.sync_copy(
        od_sh.at[pl.ds(s * STRIPE, STRIPE)],
        deg_out.at[c, 0, pl.ds(s * STRIPE, STRIPE)],
    )
    pltpu.sync_copy(
        id_sh.at[pl.ds(s * STRIPE, STRIPE)],
        deg_out.at[c, 1, pl.ds(s * STRIPE, STRIPE)],
    )


@functools.partial(
    pl.kernel,
    out_type=jax.ShapeDtypeStruct((NC, NPAD, D), jnp.float32),
    mesh=_mesh,
    scratch_types=[
        pltpu.VMEM((K,), jnp.int32),
        pltpu.VMEM((K,), jnp.int32),
        pltpu.VMEM((K, D), jnp.float32),
        pltpu.VMEM_SHARED((NPAD, D), jnp.float32),
        pltpu.SemaphoreType.DMA,
    ],
)
def _scatter_kernel(h_hbm, src_hbm, dst_hbm, agg_out, sidx, didx, rows, agg_sh, gsem):
    c = lax.axis_index("c")
    s = lax.axis_index("s")
    wid = s * NC + c

    def zbody(r, _):
        for j in range(D // 16):
            rows[r, pl.ds(j * 16, 16)] = jnp.zeros((16,), jnp.float32)
        return 0

    lax.fori_loop(0, K, zbody, 0)
    for b in range(STRIPE // K):
        pltpu.sync_copy(rows, agg_sh.at[pl.ds(s * STRIPE + b * K, K)])
    plsc.subcore_barrier()

    def body(i, _):
        base = wid * EPT + i * K
        pltpu.sync_copy(src_hbm.at[pl.ds(base, K)], sidx)
        cp = pltpu.async_copy(h_hbm.at[sidx], rows, gsem)
        pltpu.sync_copy(dst_hbm.at[pl.ds(base, K)], didx)
        cp.wait()
        pltpu.sync_copy(rows, agg_sh.at[didx], add=True)
        return 0

    lax.fori_loop(0, NCHUNK, body, 0)
    plsc.subcore_barrier()

    pltpu.sync_copy(
        agg_sh.at[pl.ds(s * STRIPE, STRIPE)],
        agg_out.at[c, pl.ds(s * STRIPE, STRIPE)],
    )


def _prep1_body(x_ref, w_ref, deg_ref, h_ref):
    d = deg_ref[...]
    on = lax.rsqrt(jnp.maximum(d[:, 0] + d[:, 2], 1.0))
    h_ref[...] = jnp.dot(
        x_ref[...] * on[:, None], w_ref[...], preferred_element_type=jnp.float32
    )


_prep1_call = pl.pallas_call(
    _prep1_body,
    grid=(N // BR,),
    in_specs=[
        pl.BlockSpec((BR, D), lambda i: (i, 0)),
        pl.BlockSpec((D, D), lambda i: (0, 0)),
        pl.BlockSpec((BR, 4), lambda i: (i, 0)),
    ],
    out_specs=pl.BlockSpec((BR, D), lambda i: (i, 0)),
    out_shape=jax.ShapeDtypeStruct((N, D), jnp.float32),
)


def _mid_body(agg_ref, deg_ref, b1_ref, w2_ref, h_ref):
    d = deg_ref[...]
    on = lax.rsqrt(jnp.maximum(d[:, 0] + d[:, 2], 1.0))
    inn = lax.rsqrt(jnp.maximum(d[:, 1] + d[:, 3], 1.0))
    a = (agg_ref[0] + agg_ref[1]) * inn[:, None] + b1_ref[...]
    h = jnp.maximum(a, 0.0) * on[:, None]
    h_ref[...] = jnp.dot(h, w2_ref[...], preferred_element_type=jnp.float32)


_mid_call = pl.pallas_call(
    _mid_body,
    grid=(N // BR,),
    in_specs=[
        pl.BlockSpec((NC, BR, D), lambda i: (0, i, 0)),
        pl.BlockSpec((BR, 4), lambda i: (i, 0)),
        pl.BlockSpec((1, D), lambda i: (0, 0)),
        pl.BlockSpec((D, D), lambda i: (0, 0)),
    ],
    out_specs=pl.BlockSpec((BR, D), lambda i: (i, 0)),
    out_shape=jax.ShapeDtypeStruct((N, D), jnp.float32),
)


def _final_body(agg_ref, deg_ref, b2_ref, out_ref):
    d = deg_ref[...]
    inn = lax.rsqrt(jnp.maximum(d[:, 1] + d[:, 3], 1.0))
    out_ref[...] = (agg_ref[0] + agg_ref[1]) * inn[:, None] + b2_ref[...]


_final_call = pl.pallas_call(
    _final_body,
    grid=(N // BR,),
    in_specs=[
        pl.BlockSpec((NC, BR, D), lambda i: (0, i, 0)),
        pl.BlockSpec((BR, 4), lambda i: (i, 0)),
        pl.BlockSpec((1, D), lambda i: (0, 0)),
    ],
    out_specs=pl.BlockSpec((BR, D), lambda i: (i, 0)),
    out_shape=jax.ShapeDtypeStruct((N, D), jnp.float32),
)


def kernel(in_feat, edge_index, W1, b1, W2, b2):
    src = edge_index[0]
    dst = edge_index[1]
    deg = _deg_kernel(src, dst)
    deg4 = deg.transpose(2, 0, 1).reshape(NPAD, 4)
    h1s = _prep1_call(in_feat, W1, deg4)
    agg1 = _scatter_kernel(h1s, src, dst)
    h2s = _mid_call(agg1, deg4, b1.reshape(1, D), W2)
    agg2 = _scatter_kernel(h2s, src, dst)
    return _final_call(agg2, deg4, b2.reshape(1, D))


# speedup vs baseline: 13.4362x; 2.3808x over previous
import functools

import jax
import jax.numpy as jnp
from jax import lax
from jax.experimental import pallas as pl
from jax.experimental.pallas import tpu as pltpu
from jax.experimental.pallas import tpu_sc as plsc

N = 10000
D = 128
E = 320000
NC = 2
NS = 16
NW = NC * NS
EPT = E // NW
KB = 100
NCH = EPT // KB
NPAD = 10240
STRIPE = NPAD // NS
BR = 1000

_mesh = plsc.VectorSubcoreMesh(
    core_axis_name="c", subcore_axis_name="s", num_cores=NC, num_subcores=NS
)


@functools.partial(
    pl.kernel,
    out_type=jax.ShapeDtypeStruct((NC, 2, NPAD), jnp.float32),
    mesh=_mesh,
    scratch_types=[
        pltpu.VMEM((NCH, KB), jnp.int32),
        pltpu.VMEM((NCH, KB), jnp.int32),
        pltpu.VMEM((128,), jnp.float32),
        pltpu.VMEM((STRIPE,), jnp.float32),
        pltpu.VMEM_SHARED((NPAD,), jnp.float32),
        pltpu.VMEM_SHARED((NPAD,), jnp.float32),
        pltpu.SemaphoreType.DMA,
        pltpu.SemaphoreType.DMA,
        pltpu.SemaphoreType.DMA,
    ],
)
def _deg_kernel(src_hbm, dst_hbm, deg_out, sidx, didx, ones_v, zb, od_sh, id_sh,
                semi, s1, s2):
    c = lax.axis_index("c")
    s = lax.axis_index("s")
    wid = s * NC + c

    cps = pltpu.async_copy(src_hbm.at[wid], sidx, semi)
    cpd = pltpu.async_copy(dst_hbm.at[wid], didx, semi)

    for j in range(8):
        ones_v[pl.ds(j * 16, 16)] = jnp.ones((16,), jnp.float32)

    def zbody(i, _):
        zb[pl.ds(i * 16, 16)] = jnp.zeros((16,), jnp.float32)
        return 0

    lax.fori_loop(0, STRIPE // 16, zbody, 0)
    pltpu.sync_copy(zb, od_sh.at[pl.ds(s * STRIPE, STRIPE)])
    pltpu.sync_copy(zb, id_sh.at[pl.ds(s * STRIPE, STRIPE)])
    cps.wait()
    cpd.wait()
    plsc.subcore_barrier()

    onesk = ones_v.at[pl.ds(0, KB)]

    def body(i, _):
        pltpu.async_copy(onesk, od_sh.at[sidx.at[i]], s1, add=True)
        pltpu.async_copy(onesk, id_sh.at[didx.at[i]], s2, add=True)
        return 0

    lax.fori_loop(0, NCH, body, 0)

    def drain(i, _):
        pltpu.make_async_copy(onesk, od_sh.at[sidx.at[0]], s1).wait()
        pltpu.make_async_copy(onesk, id_sh.at[didx.at[0]], s2).wait()
        return 0

    lax.fori_loop(0, NCH, drain, 0)
    plsc.subcore_barrier()

    pltpu.sync_copy(
        od_sh.at[pl.ds(s * STRIPE, STRIPE)],
        deg_out.at[c, 0, pl.ds(s * STRIPE, STRIPE)],
    )
    pltpu.sync_copy(
        id_sh.at[pl.ds(s * STRIPE, STRIPE)],
        deg_out.at[c, 1, pl.ds(s * STRIPE, STRIPE)],
    )


@functools.partial(
    pl.kernel,
    out_type=jax.ShapeDtypeStruct((NC, NPAD, D), jnp.float32),
    mesh=_mesh,
    scratch_types=[
        pltpu.VMEM((NCH, KB), jnp.int32),
        pltpu.VMEM((KB,), jnp.int32),
        pltpu.VMEM((KB,), jnp.int32),
        pltpu.VMEM((KB, D), jnp.float32),
        pltpu.VMEM((KB, D), jnp.float32),
        pltpu.VMEM_SHARED((NPAD, D), jnp.float32),
        pltpu.SemaphoreType.DMA,
        pltpu.SemaphoreType.DMA,
        pltpu.SemaphoreType.DMA,
        pltpu.SemaphoreType.DMA,
        pltpu.SemaphoreType.DMA,
        pltpu.SemaphoreType.DMA,
    ],
)
def _scatter_kernel(h_hbm, src_hbm, dst_hbm, agg_out, sidx, didxA, didxB,
                    rowsA, rowsB, agg_sh, semi, dA, dB, gA, gB, ssem):
    c = lax.axis_index("c")
    s = lax.axis_index("s")
    wid = s * NC + c

    cps = pltpu.async_copy(src_hbm.at[wid], sidx, semi)

    def zbody(r, _):
        for j in range(D // 16):
            rowsA[r, pl.ds(j * 16, 16)] = jnp.zeros((16,), jnp.float32)
        return 0

    lax.fori_loop(0, KB, zbody, 0)
    rz = rowsA.at[pl.ds(0, 80)]
    for b in range(STRIPE // 80):
        pltpu.sync_copy(rz, agg_sh.at[pl.ds(s * STRIPE + b * 80, 80)])
    cps.wait()
    plsc.subcore_barrier()

    pltpu.async_copy(dst_hbm.at[wid, 0], didxA, dA)
    pltpu.async_copy(h_hbm.at[sidx.at[0]], rowsA, gA)
    pltpu.async_copy(dst_hbm.at[wid, 1], didxB, dB)
    pltpu.async_copy(h_hbm.at[sidx.at[1]], rowsB, gB)

    def body(j, _):
        i0 = 2 * j
        pltpu.make_async_copy(h_hbm.at[sidx.at[0]], rowsA, gA).wait()
        pltpu.make_async_copy(dst_hbm.at[wid, 0], didxA, dA).wait()
        pltpu.async_copy(rowsA, agg_sh.at[didxA], ssem, add=True)
        pltpu.make_async_copy(rowsA, agg_sh.at[didxA], ssem).wait()

        @pl.when(j < NCH // 2 - 1)
        def _():
            pltpu.async_copy(dst_hbm.at[wid, i0 + 2], didxA, dA)
            pltpu.async_copy(h_hbm.at[sidx.at[i0 + 2]], rowsA, gA)

        pltpu.make_async_copy(h_hbm.at[sidx.at[1]], rowsB, gB).wait()
        pltpu.make_async_copy(dst_hbm.at[wid, 1], didxB, dB).wait()
        pltpu.async_copy(rowsB, agg_sh.at[didxB], ssem, add=True)
        pltpu.make_async_copy(rowsB, agg_sh.at[didxB], ssem).wait()

        @pl.when(j < NCH // 2 - 1)
        def _():
            pltpu.async_copy(dst_hbm.at[wid, i0 + 3], didxB, dB)
            pltpu.async_copy(h_hbm.at[sidx.at[i0 + 3]], rowsB, gB)

        return 0

    lax.fori_loop(0, NCH // 2, body, 0)
    plsc.subcore_barrier()

    pltpu.sync_copy(
        agg_sh.at[pl.ds(s * STRIPE, STRIPE)],
        agg_out.at[c, pl.ds(s * STRIPE, STRIPE)],
    )


def _prep1_body(x_ref, w_ref, deg_ref, h_ref):
    d = deg_ref[...]
    on = lax.rsqrt(jnp.maximum(d[:, 0] + d[:, 2], 1.0))
    h_ref[...] = jnp.dot(
        x_ref[...] * on[:, None], w_ref[...], preferred_element_type=jnp.float32
    )


_prep1_call = pl.pallas_call(
    _prep1_body,
    grid=(N // BR,),
    in_specs=[
        pl.BlockSpec((BR, D), lambda i: (i, 0)),
        pl.BlockSpec((D, D), lambda i: (0, 0)),
        pl.BlockSpec((BR, 4), lambda i: (i, 0)),
    ],
    out_specs=pl.BlockSpec((BR, D), lambda i: (i, 0)),
    out_shape=jax.ShapeDtypeStruct((N, D), jnp.float32),
)


def _mid_body(agg_ref, deg_ref, b1_ref, w2_ref, h_ref):
    d = deg_ref[...]
    on = lax.rsqrt(jnp.maximum(d[:, 0] + d[:, 2], 1.0))
    inn = lax.rsqrt(jnp.maximum(d[:, 1] + d[:, 3], 1.0))
    a = (agg_ref[0] + agg_ref[1]) * inn[:, None] + b1_ref[...]
    h = jnp.maximum(a, 0.0) * on[:, None]
    h_ref[...] = jnp.dot(h, w2_ref[...], preferred_element_type=jnp.float32)


_mid_call = pl.pallas_call(
    _mid_body,
    grid=(N // BR,),
    in_specs=[
        pl.BlockSpec((NC, BR, D), lambda i: (0, i, 0)),
        pl.BlockSpec((BR, 4), lambda i: (i, 0)),
        pl.BlockSpec((1, D), lambda i: (0, 0)),
        pl.BlockSpec((D, D), lambda i: (0, 0)),
    ],
    out_specs=pl.BlockSpec((BR, D), lambda i: (i, 0)),
    out_shape=jax.ShapeDtypeStruct((N, D), jnp.float32),
)


def _final_body(agg_ref, deg_ref, b2_ref, out_ref):
    d = deg_ref[...]
    inn = lax.rsqrt(jnp.maximum(d[:, 1] + d[:, 3], 1.0))
    out_ref[...] = (agg_ref[0] + agg_ref[1]) * inn[:, None] + b2_ref[...]


_final_call = pl.pallas_call(
    _final_body,
    grid=(N // BR,),
    in_specs=[
        pl.BlockSpec((NC, BR, D), lambda i: (0, i, 0)),
        pl.BlockSpec((BR, 4), lambda i: (i, 0)),
        pl.BlockSpec((1, D), lambda i: (0, 0)),
    ],
    out_specs=pl.BlockSpec((BR, D), lambda i: (i, 0)),
    out_shape=jax.ShapeDtypeStruct((N, D), jnp.float32),
)


def kernel(in_feat, edge_index, W1, b1, W2, b2):
    src3 = edge_index[0].reshape(NW, NCH, KB)
    dst3 = edge_index[1].reshape(NW, NCH, KB)
    deg = _deg_kernel(src3, dst3)
    deg4 = deg.transpose(2, 0, 1).reshape(NPAD, 4)
    h1s = _prep1_call(in_feat, W1, deg4)
    agg1 = _scatter_kernel(h1s, src3, dst3)
    h2s = _mid_call(agg1, deg4, b1.reshape(1, D), W2)
    agg2 = _scatter_kernel(h2s, src3, dst3)
    return _final_call(agg2, deg4, b2.reshape(1, D))


# speedup vs baseline: 13.9403x; 1.0375x over previous
import functools

import jax
import jax.numpy as jnp
from jax import lax
from jax.experimental import pallas as pl
from jax.experimental.pallas import tpu as pltpu
from jax.experimental.pallas import tpu_sc as plsc

N = 10000
D = 128
E = 320000
NC = 2
NS = 16
NW = NC * NS
EPT = E // NW
KB = 100
NCH = EPT // KB
NPAD = 10240
STRIPE = NPAD // NS
BR = 1000

_mesh = plsc.VectorSubcoreMesh(
    core_axis_name="c", subcore_axis_name="s", num_cores=NC, num_subcores=NS
)


@functools.partial(
    pl.kernel,
    out_type=jax.ShapeDtypeStruct((4, NPAD), jnp.float32),
    mesh=_mesh,
    scratch_types=[
        pltpu.VMEM((NCH, KB), jnp.int32),
        pltpu.VMEM((NCH, KB), jnp.int32),
        pltpu.VMEM((128,), jnp.float32),
        pltpu.VMEM((STRIPE,), jnp.float32),
        pltpu.VMEM_SHARED((NPAD,), jnp.float32),
        pltpu.VMEM_SHARED((NPAD,), jnp.float32),
        pltpu.SemaphoreType.DMA,
        pltpu.SemaphoreType.DMA,
        pltpu.SemaphoreType.DMA,
    ],
)
def _deg_kernel(ei_hbm, deg_out, sidx, didx, ones_v, zb, od_sh, id_sh,
                semi, s1, s2):
    c = lax.axis_index("c")
    s = lax.axis_index("s")
    wid = s * NC + c

    cps = pltpu.async_copy(ei_hbm.at[0, wid], sidx, semi)
    cpd = pltpu.async_copy(ei_hbm.at[1, wid], didx, semi)

    for j in range(8):
        ones_v[pl.ds(j * 16, 16)] = jnp.ones((16,), jnp.float32)

    def zbody(i, _):
        zb[pl.ds(i * 16, 16)] = jnp.zeros((16,), jnp.float32)
        return 0

    lax.fori_loop(0, STRIPE // 16, zbody, 0)
    pltpu.sync_copy(zb, od_sh.at[pl.ds(s * STRIPE, STRIPE)])
    pltpu.sync_copy(zb, id_sh.at[pl.ds(s * STRIPE, STRIPE)])
    cps.wait()
    cpd.wait()
    plsc.subcore_barrier()

    onesk = ones_v.at[pl.ds(0, KB)]

    def body(i, _):
        pltpu.async_copy(onesk, od_sh.at[sidx.at[i]], s1, add=True)
        pltpu.async_copy(onesk, id_sh.at[didx.at[i]], s2, add=True)
        return 0

    lax.fori_loop(0, NCH, body, 0)

    def drain(i, _):
        pltpu.make_async_copy(onesk, od_sh.at[sidx.at[0]], s1).wait()
        pltpu.make_async_copy(onesk, id_sh.at[didx.at[0]], s2).wait()
        return 0

    lax.fori_loop(0, NCH, drain, 0)
    plsc.subcore_barrier()

    pltpu.sync_copy(
        od_sh.at[pl.ds(s * STRIPE, STRIPE)],
        deg_out.at[2 * c, pl.ds(s * STRIPE, STRIPE)],
    )
    pltpu.sync_copy(
        id_sh.at[pl.ds(s * STRIPE, STRIPE)],
        deg_out.at[2 * c + 1, pl.ds(s * STRIPE, STRIPE)],
    )


@functools.partial(
    pl.kernel,
    out_type=jax.ShapeDtypeStruct((NC, NPAD, D), jnp.float32),
    mesh=_mesh,
    scratch_types=[
        pltpu.VMEM((NCH, KB), jnp.int32),
        pltpu.VMEM((KB,), jnp.int32),
        pltpu.VMEM((KB,), jnp.int32),
        pltpu.VMEM((KB, D), jnp.float32),
        pltpu.VMEM((KB, D), jnp.float32),
        pltpu.VMEM_SHARED((NPAD, D), jnp.float32),
        pltpu.SemaphoreType.DMA,
        pltpu.SemaphoreType.DMA,
        pltpu.SemaphoreType.DMA,
        pltpu.SemaphoreType.DMA,
        pltpu.SemaphoreType.DMA,
        pltpu.SemaphoreType.DMA,
    ],
)
def _scatter_kernel(h_hbm, ei_hbm, agg_out, sidx, didxA, didxB,
                    rowsA, rowsB, agg_sh, semi, dA, dB, gA, gB, ssem):
    c = lax.axis_index("c")
    s = lax.axis_index("s")
    wid = s * NC + c

    cps = pltpu.async_copy(ei_hbm.at[0, wid], sidx, semi)

    def zbody(r, _):
        for j in range(D // 16):
            rowsA[r, pl.ds(j * 16, 16)] = jnp.zeros((16,), jnp.float32)
        return 0

    lax.fori_loop(0, KB, zbody, 0)
    rz = rowsA.at[pl.ds(0, 80)]
    for b in range(STRIPE // 80):
        pltpu.sync_copy(rz, agg_sh.at[pl.ds(s * STRIPE + b * 80, 80)])
    cps.wait()
    plsc.subcore_barrier()

    pltpu.async_copy(ei_hbm.at[1, wid, 0], didxA, dA)
    pltpu.async_copy(h_hbm.at[sidx.at[0]], rowsA, gA)
    pltpu.async_copy(ei_hbm.at[1, wid, 1], didxB, dB)
    pltpu.async_copy(h_hbm.at[sidx.at[1]], rowsB, gB)

    def body(j, _):
        i0 = 2 * j
        pltpu.make_async_copy(h_hbm.at[sidx.at[0]], rowsA, gA).wait()
        pltpu.make_async_copy(ei_hbm.at[1, wid, 0], didxA, dA).wait()
        pltpu.async_copy(rowsA, agg_sh.at[didxA], ssem, add=True)
        pltpu.make_async_copy(rowsA, agg_sh.at[didxA], ssem).wait()

        @pl.when(j < NCH // 2 - 1)
        def _():
            pltpu.async_copy(ei_hbm.at[1, wid, i0 + 2], didxA, dA)
            pltpu.async_copy(h_hbm.at[sidx.at[i0 + 2]], rowsA, gA)

        pltpu.make_async_copy(h_hbm.at[sidx.at[1]], rowsB, gB).wait()
        pltpu.make_async_copy(ei_hbm.at[1, wid, 1], didxB, dB).wait()
        pltpu.async_copy(rowsB, agg_sh.at[didxB], ssem, add=True)
        pltpu.make_async_copy(rowsB, agg_sh.at[didxB], ssem).wait()

        @pl.when(j < NCH // 2 - 1)
        def _():
            pltpu.async_copy(ei_hbm.at[1, wid, i0 + 3], didxB, dB)
            pltpu.async_copy(h_hbm.at[sidx.at[i0 + 3]], rowsB, gB)

        return 0

    lax.fori_loop(0, NCH // 2, body, 0)
    plsc.subcore_barrier()

    pltpu.sync_copy(
        agg_sh.at[pl.ds(s * STRIPE, STRIPE)],
        agg_out.at[c, pl.ds(s * STRIPE, STRIPE)],
    )


def _mm1_body(x_ref, w_ref, u_ref):
    u_ref[...] = jnp.dot(x_ref[...], w_ref[...], preferred_element_type=jnp.float32)


_mm1_call = pl.pallas_call(
    _mm1_body,
    grid=(N // BR,),
    in_specs=[
        pl.BlockSpec((BR, D), lambda i: (i, 0)),
        pl.BlockSpec((D, D), lambda i: (0, 0)),
    ],
    out_specs=pl.BlockSpec((BR, D), lambda i: (i, 0)),
    out_shape=jax.ShapeDtypeStruct((N, D), jnp.float32),
)


def _scale_body(u_ref, deg_ref, h_ref):
    d = deg_ref[...]
    on = lax.rsqrt(jnp.maximum(d[:, 0] + d[:, 2], 1.0))
    h_ref[...] = u_ref[...] * on[:, None]


_scale_call = pl.pallas_call(
    _scale_body,
    grid=(N // BR,),
    in_specs=[
        pl.BlockSpec((BR, D), lambda i: (i, 0)),
        pl.BlockSpec((BR, 4), lambda i: (i, 0)),
    ],
    out_specs=pl.BlockSpec((BR, D), lambda i: (i, 0)),
    out_shape=jax.ShapeDtypeStruct((N, D), jnp.float32),
)


def _mid_body(agg_ref, deg_ref, b1_ref, w2_ref, h_ref):
    d = deg_ref[...]
    on = lax.rsqrt(jnp.maximum(d[:, 0] + d[:, 2], 1.0))
    inn = lax.rsqrt(jnp.maximum(d[:, 1] + d[:, 3], 1.0))
    a = (agg_ref[0] + agg_ref[1]) * inn[:, None] + b1_ref[...]
    h = jnp.maximum(a, 0.0) * on[:, None]
    h_ref[...] = jnp.dot(h, w2_ref[...], preferred_element_type=jnp.float32)


_mid_call = pl.pallas_call(
    _mid_body,
    grid=(N // BR,),
    in_specs=[
        pl.BlockSpec((NC, BR, D), lambda i: (0, i, 0)),
        pl.BlockSpec((BR, 4), lambda i: (i, 0)),
        pl.BlockSpec((1, D), lambda i: (0, 0)),
        pl.BlockSpec((D, D), lambda i: (0, 0)),
    ],
    out_specs=pl.BlockSpec((BR, D), lambda i: (i, 0)),
    out_shape=jax.ShapeDtypeStruct((N, D), jnp.float32),
)


def _final_body(agg_ref, deg_ref, b2_ref, out_ref):
    d = deg_ref[...]
    inn = lax.rsqrt(jnp.maximum(d[:, 1] + d[:, 3], 1.0))
    out_ref[...] = (agg_ref[0] + agg_ref[1]) * inn[:, None] + b2_ref[...]


_final_call = pl.pallas_call(
    _final_body,
    grid=(N // BR,),
    in_specs=[
        pl.BlockSpec((NC, BR, D), lambda i: (0, i, 0)),
        pl.BlockSpec((BR, 4), lambda i: (i, 0)),
        pl.BlockSpec((1, D), lambda i: (0, 0)),
    ],
    out_specs=pl.BlockSpec((BR, D), lambda i: (i, 0)),
    out_shape=jax.ShapeDtypeStruct((N, D), jnp.float32),
)


def kernel(in_feat, edge_index, W1, b1, W2, b2):
    ei4 = edge_index.reshape(2, NW, NCH, KB)
    degT = _deg_kernel(ei4)
    u1 = _mm1_call(in_feat, W1)
    deg = degT.T
    h1s = _scale_call(u1, deg)
    agg1 = _scatter_kernel(h1s, ei4)
    h2s = _mid_call(agg1, deg, b1.reshape(1, D), W2)
    agg2 = _scatter_kernel(h2s, ei4)
    return _final_call(agg2, deg, b2.reshape(1, D))


# speedup vs baseline: 14.2538x; 1.0225x over previous
import functools

import jax
import jax.numpy as jnp
from jax import lax
from jax.experimental import pallas as pl
from jax.experimental.pallas import tpu as pltpu
from jax.experimental.pallas import tpu_sc as plsc

N = 10000
D = 128
E = 320000
NC = 2
NS = 16
NW = NC * NS
EPT = E // NW
KB = 100
NCH = EPT // KB
NPAD = 10240
STRIPE = NPAD // NS
BR = 2000

_mesh = plsc.VectorSubcoreMesh(
    core_axis_name="c", subcore_axis_name="s", num_cores=NC, num_subcores=NS
)


@functools.partial(
    pl.kernel,
    out_type=jax.ShapeDtypeStruct((4, NPAD), jnp.float32),
    mesh=_mesh,
    scratch_types=[
        pltpu.VMEM((NCH, KB), jnp.int32),
        pltpu.VMEM((NCH, KB), jnp.int32),
        pltpu.VMEM((128,), jnp.float32),
        pltpu.VMEM((STRIPE,), jnp.float32),
        pltpu.VMEM_SHARED((NPAD,), jnp.float32),
        pltpu.VMEM_SHARED((NPAD,), jnp.float32),
        pltpu.SemaphoreType.DMA,
        pltpu.SemaphoreType.DMA,
        pltpu.SemaphoreType.DMA,
    ],
)
def _deg_kernel(ei_hbm, deg_out, sidx, didx, ones_v, zb, od_sh, id_sh,
                semi, s1, s2):
    c = lax.axis_index("c")
    s = lax.axis_index("s")
    wid = s * NC + c

    cps = pltpu.async_copy(ei_hbm.at[0, wid], sidx, semi)
    cpd = pltpu.async_copy(ei_hbm.at[1, wid], didx, semi)

    for j in range(8):
        ones_v[pl.ds(j * 16, 16)] = jnp.ones((16,), jnp.float32)

    def zbody(i, _):
        zb[pl.ds(i * 16, 16)] = jnp.zeros((16,), jnp.float32)
        return 0

    lax.fori_loop(0, STRIPE // 16, zbody, 0)
    pltpu.sync_copy(zb, od_sh.at[pl.ds(s * STRIPE, STRIPE)])
    pltpu.sync_copy(zb, id_sh.at[pl.ds(s * STRIPE, STRIPE)])
    cps.wait()
    cpd.wait()
    plsc.subcore_barrier()

    onesk = ones_v.at[pl.ds(0, KB)]

    def body(i, _):
        pltpu.async_copy(onesk, od_sh.at[sidx.at[i]], s1, add=True)
        pltpu.async_copy(onesk, id_sh.at[didx.at[i]], s2, add=True)
        return 0

    lax.fori_loop(0, NCH, body, 0)

    def drain(i, _):
        pltpu.make_async_copy(onesk, od_sh.at[sidx.at[0]], s1).wait()
        pltpu.make_async_copy(onesk, id_sh.at[didx.at[0]], s2).wait()
        return 0

    lax.fori_loop(0, NCH, drain, 0)
    plsc.subcore_barrier()

    pltpu.sync_copy(
        od_sh.at[pl.ds(s * STRIPE, STRIPE)],
        deg_out.at[2 * c, pl.ds(s * STRIPE, STRIPE)],
    )
    pltpu.sync_copy(
        id_sh.at[pl.ds(s * STRIPE, STRIPE)],
        deg_out.at[2 * c + 1, pl.ds(s * STRIPE, STRIPE)],
    )


@functools.partial(
    pl.kernel,
    out_type=jax.ShapeDtypeStruct((NC, NPAD, D), jnp.float32),
    mesh=_mesh,
    scratch_types=[
        pltpu.VMEM((NCH, KB), jnp.int32),
        pltpu.VMEM((KB,), jnp.int32),
        pltpu.VMEM((KB,), jnp.int32),
        pltpu.VMEM((KB, D), jnp.float32),
        pltpu.VMEM((KB, D), jnp.float32),
        pltpu.VMEM_SHARED((NPAD, D), jnp.float32),
        pltpu.SemaphoreType.DMA,
        pltpu.SemaphoreType.DMA,
        pltpu.SemaphoreType.DMA,
        pltpu.SemaphoreType.DMA,
        pltpu.SemaphoreType.DMA,
        pltpu.SemaphoreType.DMA,
    ],
)
def _scatter_kernel(h_hbm, ei_hbm, agg_out, sidx, didxA, didxB,
                    rowsA, rowsB, agg_sh, semi, dA, dB, gA, gB, ssem):
    c = lax.axis_index("c")
    s = lax.axis_index("s")
    wid = s * NC + c

    cps = pltpu.async_copy(ei_hbm.at[0, wid], sidx, semi)

    def zbody(r, _):
        for j in range(D // 16):
            rowsA[r, pl.ds(j * 16, 16)] = jnp.zeros((16,), jnp.float32)
        return 0

    lax.fori_loop(0, KB, zbody, 0)
    rz = rowsA.at[pl.ds(0, 80)]
    for b in range(STRIPE // 80):
        pltpu.sync_copy(rz, agg_sh.at[pl.ds(s * STRIPE + b * 80, 80)])
    cps.wait()
    plsc.subcore_barrier()

    pltpu.async_copy(ei_hbm.at[1, wid, 0], didxA, dA)
    pltpu.async_copy(h_hbm.at[sidx.at[0]], rowsA, gA)
    pltpu.async_copy(ei_hbm.at[1, wid, 1], didxB, dB)
    pltpu.async_copy(h_hbm.at[sidx.at[1]], rowsB, gB)

    def body(j, _):
        i0 = 2 * j
        pltpu.make_async_copy(h_hbm.at[sidx.at[0]], rowsA, gA).wait()
        pltpu.make_async_copy(ei_hbm.at[1, wid, 0], didxA, dA).wait()
        pltpu.async_copy(rowsA, agg_sh.at[didxA], ssem, add=True)
        pltpu.make_async_copy(rowsA, agg_sh.at[didxA], ssem).wait()

        @pl.when(j < NCH // 2 - 1)
        def _():
            pltpu.async_copy(ei_hbm.at[1, wid, i0 + 2], didxA, dA)
            pltpu.async_copy(h_hbm.at[sidx.at[i0 + 2]], rowsA, gA)

        pltpu.make_async_copy(h_hbm.at[sidx.at[1]], rowsB, gB).wait()
        pltpu.make_async_copy(ei_hbm.at[1, wid, 1], didxB, dB).wait()
        pltpu.async_copy(rowsB, agg_sh.at[didxB], ssem, add=True)
        pltpu.make_async_copy(rowsB, agg_sh.at[didxB], ssem).wait()

        @pl.when(j < NCH // 2 - 1)
        def _():
            pltpu.async_copy(ei_hbm.at[1, wid, i0 + 3], didxB, dB)
            pltpu.async_copy(h_hbm.at[sidx.at[i0 + 3]], rowsB, gB)

        return 0

    lax.fori_loop(0, NCH // 2, body, 0)
    plsc.subcore_barrier()

    pltpu.sync_copy(
        agg_sh.at[pl.ds(s * STRIPE, STRIPE)],
        agg_out.at[c, pl.ds(s * STRIPE, STRIPE)],
    )


def _mm1_body(x_ref, w_ref, u_ref):
    u_ref[...] = jnp.dot(x_ref[...], w_ref[...], preferred_element_type=jnp.float32)


_mm1_call = pl.pallas_call(
    _mm1_body,
    grid=(N // BR,),
    in_specs=[
        pl.BlockSpec((BR, D), lambda i: (i, 0)),
        pl.BlockSpec((D, D), lambda i: (0, 0)),
    ],
    out_specs=pl.BlockSpec((BR, D), lambda i: (i, 0)),
    out_shape=jax.ShapeDtypeStruct((N, D), jnp.float32),
)


def _scale_body(u_ref, deg_ref, h_ref):
    d = deg_ref[...]
    on = lax.rsqrt(jnp.maximum(d[:, 0] + d[:, 2], 1.0))
    h_ref[...] = u_ref[...] * on[:, None]


_scale_call = pl.pallas_call(
    _scale_body,
    grid=(N // BR,),
    in_specs=[
        pl.BlockSpec((BR, D), lambda i: (i, 0)),
        pl.BlockSpec((BR, 4), lambda i: (i, 0)),
    ],
    out_specs=pl.BlockSpec((BR, D), lambda i: (i, 0)),
    out_shape=jax.ShapeDtypeStruct((N, D), jnp.float32),
)


def _mid_body(agg_ref, deg_ref, b1_ref, w2_ref, h_ref):
    d = deg_ref[...]
    on = lax.rsqrt(jnp.maximum(d[:, 0] + d[:, 2], 1.0))
    inn = lax.rsqrt(jnp.maximum(d[:, 1] + d[:, 3], 1.0))
    a = (agg_ref[0] + agg_ref[1]) * inn[:, None] + b1_ref[...]
    h = jnp.maximum(a, 0.0) * on[:, None]
    h_ref[...] = jnp.dot(h, w2_ref[...], preferred_element_type=jnp.float32)


_mid_call = pl.pallas_call(
    _mid_body,
    grid=(N // BR,),
    in_specs=[
        pl.BlockSpec((NC, BR, D), lambda i: (0, i, 0)),
        pl.BlockSpec((BR, 4), lambda i: (i, 0)),
        pl.BlockSpec((1, D), lambda i: (0, 0)),
        pl.BlockSpec((D, D), lambda i: (0, 0)),
    ],
    out_specs=pl.BlockSpec((BR, D), lambda i: (i, 0)),
    out_shape=jax.ShapeDtypeStruct((N, D), jnp.float32),
)


def _final_body(agg_ref, deg_ref, b2_ref, out_ref):
    d = deg_ref[...]
    inn = lax.rsqrt(jnp.maximum(d[:, 1] + d[:, 3], 1.0))
    out_ref[...] = (agg_ref[0] + agg_ref[1]) * inn[:, None] + b2_ref[...]


_final_call = pl.pallas_call(
    _final_body,
    grid=(N // BR,),
    in_specs=[
        pl.BlockSpec((NC, BR, D), lambda i: (0, i, 0)),
        pl.BlockSpec((BR, 4), lambda i: (i, 0)),
        pl.BlockSpec((1, D), lambda i: (0, 0)),
    ],
    out_specs=pl.BlockSpec((BR, D), lambda i: (i, 0)),
    out_shape=jax.ShapeDtypeStruct((N, D), jnp.float32),
)


def kernel(in_feat, edge_index, W1, b1, W2, b2):
    ei4 = edge_index.reshape(2, NW, NCH, KB)
    degT = _deg_kernel(ei4)
    u1 = _mm1_call(in_feat, W1)
    deg = degT.T
    h1s = _scale_call(u1, deg)
    agg1 = _scatter_kernel(h1s, ei4)
    h2s = _mid_call(agg1, deg, b1.reshape(1, D), W2)
    agg2 = _scatter_kernel(h2s, ei4)
    return _final_call(agg2, deg, b2.reshape(1, D))


# speedup vs baseline: 14.6059x; 1.0247x over previous
import functools

import jax
import jax.numpy as jnp
from jax import lax
from jax.experimental import pallas as pl
from jax.experimental.pallas import tpu as pltpu
from jax.experimental.pallas import tpu_sc as plsc

N = 10000
D = 128
E = 320000
NC = 2
NS = 16
NW = NC * NS
EPT = E // NW
KB = 125
NCH = EPT // KB
NPAD = 10240
STRIPE = NPAD // NS
BR = 2000

_mesh = plsc.VectorSubcoreMesh(
    core_axis_name="c", subcore_axis_name="s", num_cores=NC, num_subcores=NS
)


@functools.partial(
    pl.kernel,
    out_type=jax.ShapeDtypeStruct((4, NPAD), jnp.float32),
    mesh=_mesh,
    scratch_types=[
        pltpu.VMEM((NCH, KB), jnp.int32),
        pltpu.VMEM((NCH, KB), jnp.int32),
        pltpu.VMEM((128,), jnp.float32),
        pltpu.VMEM((STRIPE,), jnp.float32),
        pltpu.VMEM_SHARED((NPAD,), jnp.float32),
        pltpu.VMEM_SHARED((NPAD,), jnp.float32),
        pltpu.SemaphoreType.DMA,
        pltpu.SemaphoreType.DMA,
        pltpu.SemaphoreType.DMA,
    ],
)
def _deg_kernel(ei_hbm, deg_out, sidx, didx, ones_v, zb, od_sh, id_sh,
                semi, s1, s2):
    c = lax.axis_index("c")
    s = lax.axis_index("s")
    wid = s * NC + c

    cps = pltpu.async_copy(ei_hbm.at[0, wid], sidx, semi)
    cpd = pltpu.async_copy(ei_hbm.at[1, wid], didx, semi)

    for j in range(8):
        ones_v[pl.ds(j * 16, 16)] = jnp.ones((16,), jnp.float32)

    def zbody(i, _):
        zb[pl.ds(i * 16, 16)] = jnp.zeros((16,), jnp.float32)
        return 0

    lax.fori_loop(0, STRIPE // 16, zbody, 0)
    pltpu.sync_copy(zb, od_sh.at[pl.ds(s * STRIPE, STRIPE)])
    pltpu.sync_copy(zb, id_sh.at[pl.ds(s * STRIPE, STRIPE)])
    cps.wait()
    cpd.wait()
    plsc.subcore_barrier()

    onesk = ones_v.at[pl.ds(0, KB)]

    def body(i, _):
        pltpu.async_copy(onesk, od_sh.at[sidx.at[i]], s1, add=True)
        pltpu.async_copy(onesk, id_sh.at[didx.at[i]], s2, add=True)
        return 0

    lax.fori_loop(0, NCH, body, 0)

    def drain(i, _):
        pltpu.make_async_copy(onesk, od_sh.at[sidx.at[0]], s1).wait()
        pltpu.make_async_copy(onesk, id_sh.at[didx.at[0]], s2).wait()
        return 0

    lax.fori_loop(0, NCH, drain, 0)
    plsc.subcore_barrier()

    pltpu.sync_copy(
        od_sh.at[pl.ds(s * STRIPE, STRIPE)],
        deg_out.at[2 * c, pl.ds(s * STRIPE, STRIPE)],
    )
    pltpu.sync_copy(
        id_sh.at[pl.ds(s * STRIPE, STRIPE)],
        deg_out.at[2 * c + 1, pl.ds(s * STRIPE, STRIPE)],
    )


@functools.partial(
    pl.kernel,
    out_type=jax.ShapeDtypeStruct((NC, NPAD, D), jnp.float32),
    mesh=_mesh,
    scratch_types=[
        pltpu.VMEM((NCH, KB), jnp.int32),
        pltpu.VMEM((KB,), jnp.int32),
        pltpu.VMEM((KB,), jnp.int32),
        pltpu.VMEM((KB, D), jnp.float32),
        pltpu.VMEM((KB, D), jnp.float32),
        pltpu.VMEM_SHARED((NPAD, D), jnp.float32),
        pltpu.SemaphoreType.DMA,
        pltpu.SemaphoreType.DMA,
        pltpu.SemaphoreType.DMA,
        pltpu.SemaphoreType.DMA,
        pltpu.SemaphoreType.DMA,
        pltpu.SemaphoreType.DMA,
    ],
)
def _scatter_kernel(h_hbm, ei_hbm, agg_out, sidx, didxA, didxB,
                    rowsA, rowsB, agg_sh, semi, dA, dB, gA, gB, ssem):
    c = lax.axis_index("c")
    s = lax.axis_index("s")
    wid = s * NC + c

    cps = pltpu.async_copy(ei_hbm.at[0, wid], sidx, semi)

    def zbody(r, _):
        for j in range(D // 16):
            rowsA[r, pl.ds(j * 16, 16)] = jnp.zeros((16,), jnp.float32)
        return 0

    lax.fori_loop(0, KB, zbody, 0)
    rz = rowsA.at[pl.ds(0, 80)]
    for b in range(STRIPE // 80):
        pltpu.sync_copy(rz, agg_sh.at[pl.ds(s * STRIPE + b * 80, 80)])
    cps.wait()
    plsc.subcore_barrier()

    pltpu.async_copy(ei_hbm.at[1, wid, 0], didxA, dA)
    pltpu.async_copy(h_hbm.at[sidx.at[0]], rowsA, gA)
    pltpu.async_copy(ei_hbm.at[1, wid, 1], didxB, dB)
    pltpu.async_copy(h_hbm.at[sidx.at[1]], rowsB, gB)

    def body(j, _):
        i0 = 2 * j
        pltpu.make_async_copy(h_hbm.at[sidx.at[0]], rowsA, gA).wait()
        pltpu.make_async_copy(ei_hbm.at[1, wid, 0], didxA, dA).wait()
        pltpu.async_copy(rowsA, agg_sh.at[didxA], ssem, add=True)
        pltpu.make_async_copy(rowsA, agg_sh.at[didxA], ssem).wait()

        @pl.when(j < NCH // 2 - 1)
        def _():
            pltpu.async_copy(ei_hbm.at[1, wid, i0 + 2], didxA, dA)
            pltpu.async_copy(h_hbm.at[sidx.at[i0 + 2]], rowsA, gA)

        pltpu.make_async_copy(h_hbm.at[sidx.at[1]], rowsB, gB).wait()
        pltpu.make_async_copy(ei_hbm.at[1, wid, 1], didxB, dB).wait()
        pltpu.async_copy(rowsB, agg_sh.at[didxB], ssem, add=True)
        pltpu.make_async_copy(rowsB, agg_sh.at[didxB], ssem).wait()

        @pl.when(j < NCH // 2 - 1)
        def _():
            pltpu.async_copy(ei_hbm.at[1, wid, i0 + 3], didxB, dB)
            pltpu.async_copy(h_hbm.at[sidx.at[i0 + 3]], rowsB, gB)

        return 0

    lax.fori_loop(0, NCH // 2, body, 0)
    plsc.subcore_barrier()

    pltpu.sync_copy(
        agg_sh.at[pl.ds(s * STRIPE, STRIPE)],
        agg_out.at[c, pl.ds(s * STRIPE, STRIPE)],
    )


def _mm1_body(x_ref, w_ref, u_ref):
    u_ref[...] = jnp.dot(x_ref[...], w_ref[...], preferred_element_type=jnp.float32)


_mm1_call = pl.pallas_call(
    _mm1_body,
    grid=(N // BR,),
    in_specs=[
        pl.BlockSpec((BR, D), lambda i: (i, 0)),
        pl.BlockSpec((D, D), lambda i: (0, 0)),
    ],
    out_specs=pl.BlockSpec((BR, D), lambda i: (i, 0)),
    out_shape=jax.ShapeDtypeStruct((N, D), jnp.float32),
)


def _scale_body(u_ref, deg_ref, h_ref):
    d = deg_ref[...]
    on = lax.rsqrt(jnp.maximum(d[:, 0] + d[:, 2], 1.0))
    h_ref[...] = u_ref[...] * on[:, None]


_scale_call = pl.pallas_call(
    _scale_body,
    grid=(N // BR,),
    in_specs=[
        pl.BlockSpec((BR, D), lambda i: (i, 0)),
        pl.BlockSpec((BR, 4), lambda i: (i, 0)),
    ],
    out_specs=pl.BlockSpec((BR, D), lambda i: (i, 0)),
    out_shape=jax.ShapeDtypeStruct((N, D), jnp.float32),
)


def _mid_body(agg_ref, deg_ref, b1_ref, w2_ref, h_ref):
    d = deg_ref[...]
    on = lax.rsqrt(jnp.maximum(d[:, 0] + d[:, 2], 1.0))
    inn = lax.rsqrt(jnp.maximum(d[:, 1] + d[:, 3], 1.0))
    a = (agg_ref[0] + agg_ref[1]) * inn[:, None] + b1_ref[...]
    h = jnp.maximum(a, 0.0) * on[:, None]
    h_ref[...] = jnp.dot(h, w2_ref[...], preferred_element_type=jnp.float32)


_mid_call = pl.pallas_call(
    _mid_body,
    grid=(N // BR,),
    in_specs=[
        pl.BlockSpec((NC, BR, D), lambda i: (0, i, 0)),
        pl.BlockSpec((BR, 4), lambda i: (i, 0)),
        pl.BlockSpec((1, D), lambda i: (0, 0)),
        pl.BlockSpec((D, D), lambda i: (0, 0)),
    ],
    out_specs=pl.BlockSpec((BR, D), lambda i: (i, 0)),
    out_shape=jax.ShapeDtypeStruct((N, D), jnp.float32),
)


def _final_body(agg_ref, deg_ref, b2_ref, out_ref):
    d = deg_ref[...]
    inn = lax.rsqrt(jnp.maximum(d[:, 1] + d[:, 3], 1.0))
    out_ref[...] = (agg_ref[0] + agg_ref[1]) * inn[:, None] + b2_ref[...]


_final_call = pl.pallas_call(
    _final_body,
    grid=(N // BR,),
    in_specs=[
        pl.BlockSpec((NC, BR, D), lambda i: (0, i, 0)),
        pl.BlockSpec((BR, 4), lambda i: (i, 0)),
        pl.BlockSpec((1, D), lambda i: (0, 0)),
    ],
    out_specs=pl.BlockSpec((BR, D), lambda i: (i, 0)),
    out_shape=jax.ShapeDtypeStruct((N, D), jnp.float32),
)


def kernel(in_feat, edge_index, W1, b1, W2, b2):
    ei4 = edge_index.reshape(2, NW, NCH, KB)
    degT = _deg_kernel(ei4)
    u1 = _mm1_call(in_feat, W1)
    deg = degT.T
    h1s = _scale_call(u1, deg)
    agg1 = _scatter_kernel(h1s, ei4)
    h2s = _mid_call(agg1, deg, b1.reshape(1, D), W2)
    agg2 = _scatter_kernel(h2s, ei4)
    return _final_call(agg2, deg, b2.reshape(1, D))
